# Initial kernel scaffold; baseline (speedup 1.0000x reference)
import jax
import jax.numpy as jnp
from jax.experimental import pallas as pl


def kernel(x, w1, b1, w2, b2):
    raise NotImplementedError("write your pallas kernel here")


# speedup vs baseline: 4.7693x; 4.7693x over previous
import jax
import jax.numpy as jnp
from jax.experimental import pallas as pl
from jax.experimental.pallas import tpu as pltpu

_N_BISECT = 28
_BM = 1024


def _fused_body(x_ref, w1_ref, b1_ref, w2_ref, b2_ref, eye_ref, o_ref):
    h = jnp.dot(x_ref[...], w1_ref[...], preferred_element_type=jnp.float32)
    h = jnp.maximum(h + b1_ref[...], 0.0)
    us = jax.lax.dot_general(
        w2_ref[...], h, (((0,), (1,)), ((), ())),
        preferred_element_type=jnp.float32,
    ) + b2_ref[...]

    lo = jnp.min(us, axis=0, keepdims=True) - 10.0
    hi = jnp.max(us, axis=0, keepdims=True)
    for _ in range(_N_BISECT):
        mid = 0.5 * (lo + hi)
        f = jnp.sum(jnp.maximum(us - mid, 0.0), axis=0, keepdims=True) - 1.0
        pos = f > 0.0
        lo = jnp.where(pos, mid, lo)
        hi = jnp.where(pos, hi, mid)
    tau = 0.5 * (lo + hi)

    p = jnp.maximum(us - tau, 0.0)
    p = p / (jnp.sum(p, axis=0, keepdims=True) + 1e-8)
    o_ref[...] = jax.lax.dot_general(
        p, eye_ref[...], (((0,), (0,)), ((), ())),
        preferred_element_type=jnp.float32,
    )


def kernel(x, w1, b1, w2, b2):
    B, D = x.shape
    H = w1.shape[1]
    E = w2.shape[1]
    b1_2d = b1.reshape(1, H).astype(jnp.float32)
    b2_2d = b2.reshape(E, 1).astype(jnp.float32)
    eye = jnp.eye(E, dtype=jnp.float32)
    return pl.pallas_call(
        _fused_body,
        out_shape=jax.ShapeDtypeStruct((B, E), jnp.float32),
        grid=(B // _BM,),
        in_specs=[
            pl.BlockSpec((_BM, D), lambda i: (i, 0)),
            pl.BlockSpec((D, H), lambda i: (0, 0)),
            pl.BlockSpec((1, H), lambda i: (0, 0)),
            pl.BlockSpec((H, E), lambda i: (0, 0)),
            pl.BlockSpec((E, 1), lambda i: (0, 0)),
            pl.BlockSpec((E, E), lambda i: (0, 0)),
        ],
        out_specs=pl.BlockSpec((_BM, E), lambda i: (i, 0)),
        compiler_params=pltpu.CompilerParams(
            dimension_semantics=("parallel",),
            vmem_limit_bytes=50 * 1024 * 1024,
        ),
        name="tsallis_router_fused",
    )(x, w1, b1_2d, w2, b2_2d, eye)
